# Initial kernel scaffold; baseline (speedup 1.0000x reference)
import jax
import jax.numpy as jnp
from jax.experimental import pallas as pl


def kernel(x, edge_index, W1, b1, W2, b2):
    raise NotImplementedError("write your pallas kernel here")


# speedup vs baseline: 11.0536x; 11.0536x over previous
import functools

import jax
import jax.numpy as jnp
from jax import lax
from jax.experimental import pallas as pl
from jax.experimental.pallas import tpu as pltpu
from jax.experimental.pallas import tpu_sc as plsc

N_NODES = 10000
N_EDGES = 320000
D_IN, D_HID, D_OUT = 128, 256, 128

LANES = 16
N_CORES = 2
N_SUB = 16
EB = 128
NA = 10112
DUMMY = N_NODES
RZ = NA // N_SUB
OT = 10
RO = N_NODES // OT

BLK = 1000


def _round_up(v, m):
    return (v + m - 1) // m * m


def _zero_fill(buf, nrows, width):
    zero16 = jnp.zeros((LANES,), jnp.float32)

    def row(i, c):
        for j in range(width // LANES):
            buf[i, pl.ds(j * LANES, LANES)] = zero16
        return c

    lax.fori_loop(0, nrows, row, 0)


def _copy_zeros(src_buf, acc, base, total):
    off = 0
    while off < total:
        cnt = min(EB, total - off)
        pltpu.sync_copy(src_buf.at[pl.ds(0, cnt)], acc.at[pl.ds(base + off, cnt)])
        off += cnt


def _make_segsum(n_table_rows, edges_per_tile):
    nb = edges_per_tile // EB
    mesh = plsc.VectorSubcoreMesh(core_axis_name="c", subcore_axis_name="s")

    @functools.partial(
        pl.kernel,
        mesh=mesh,
        out_type=jax.ShapeDtypeStruct((N_CORES * N_NODES, 128), jnp.float32),
        scratch_types=[
            pltpu.VMEM((EB,), jnp.int32),
            pltpu.VMEM((EB,), jnp.int32),
            pltpu.VMEM((EB, 128), jnp.float32),
            pltpu.VMEM_SHARED((NA, 128), jnp.float32),
            pltpu.SemaphoreType.DMA,
        ],
    )
    def seg(table, srcr, dstr, out, sidx, didx, rows, acc, sem):
        cid = lax.axis_index("c")
        sid = lax.axis_index("s")
        wid = cid * N_SUB + sid

        _zero_fill(rows, EB, 128)
        _copy_zeros(rows, acc, sid * RZ, RZ)
        plsc.subcore_barrier()

        ebase = wid * edges_per_tile

        def body(j, c):
            b0 = ebase + j * EB
            pltpu.sync_copy(srcr.at[pl.ds(b0, EB)], sidx)
            pltpu.sync_copy(dstr.at[pl.ds(b0, EB)], didx)
            pltpu.async_copy(table.at[sidx], rows, sem).wait()
            pltpu.sync_copy(rows, acc.at[didx], add=True)
            return c

        lax.fori_loop(0, nb, body, 0)
        plsc.subcore_barrier()

        @pl.when(sid < OT)
        def _():
            obase = sid * RO
            pltpu.sync_copy(
                acc.at[pl.ds(obase, RO)],
                out.at[pl.ds(cid * N_NODES + obase, RO)],
            )

    return seg


def _make_degree(edges_per_tile):
    nb = edges_per_tile // EB
    mesh = plsc.VectorSubcoreMesh(core_axis_name="c", subcore_axis_name="s")

    @functools.partial(
        pl.kernel,
        mesh=mesh,
        out_type=jax.ShapeDtypeStruct((N_CORES * N_NODES, LANES), jnp.float32),
        scratch_types=[
            pltpu.VMEM((EB,), jnp.int32),
            pltpu.VMEM((EB, LANES), jnp.float32),
            pltpu.VMEM_SHARED((NA, LANES), jnp.float32),
        ],
    )
    def deg(dstr, out, didx, ones, acc):
        cid = lax.axis_index("c")
        sid = lax.axis_index("s")
        wid = cid * N_SUB + sid

        _zero_fill(ones, EB, LANES)
        _copy_zeros(ones, acc, sid * RZ, RZ)

        one16 = jnp.ones((LANES,), jnp.float32)

        def orow(i, c):
            ones[i, :] = one16
            return c

        lax.fori_loop(0, EB, orow, 0)
        plsc.subcore_barrier()

        ebase = wid * edges_per_tile

        def body(j, c):
            b0 = ebase + j * EB
            pltpu.sync_copy(dstr.at[pl.ds(b0, EB)], didx)
            pltpu.sync_copy(ones, acc.at[didx], add=True)
            return c

        lax.fori_loop(0, nb, body, 0)
        plsc.subcore_barrier()

        @pl.when(sid < OT)
        def _():
            obase = sid * RO
            pltpu.sync_copy(
                acc.at[pl.ds(obase, RO)],
                out.at[pl.ds(cid * N_NODES + obase, RO)],
            )

    return deg


def _deg_from_ref(degr):
    return degr[0, :, 0] + degr[1, :, 0] + 1.0


def _mm1_body(degr, xr, wr, hs_r, hself_r):
    deg = _deg_from_ref(degr)
    dis = lax.rsqrt(deg)
    h = jnp.dot(xr[...], wr[...], preferred_element_type=jnp.float32)
    hs = h * dis[:, None]
    hs_r[0] = hs[:, :128]
    hs_r[1] = hs[:, 128:]
    hself_r[...] = h * (1.0 / deg)[:, None]


def _mm2_body(degr, accr, hselfr, b1r, w2r, hs2_r, hself2_r):
    deg = _deg_from_ref(degr)
    dis = lax.rsqrt(deg)
    acc = jnp.concatenate([accr[0], accr[1]], axis=-1)
    z = jnp.maximum(acc * dis[:, None] + hselfr[...] + b1r[...], 0.0)
    h2 = jnp.dot(z, w2r[...], preferred_element_type=jnp.float32)
    hs2_r[...] = h2 * dis[:, None]
    hself2_r[...] = h2 * (1.0 / deg)[:, None]


def _final_body(degr, accr, hself2r, b2r, outr):
    deg = _deg_from_ref(degr)
    dis = lax.rsqrt(deg)
    acc = accr[0] + accr[1]
    outr[...] = jnp.maximum(acc * dis[:, None] + hself2r[...] + b2r[...], 0.0)


def _deg_spec():
    return pl.BlockSpec((2, BLK, LANES), lambda i: (0, i, 0))


def _row_spec(d):
    return pl.BlockSpec((BLK, d), lambda i: (i, 0))


def _full_spec(a, b):
    return pl.BlockSpec((a, b), lambda i: (0, 0))


def _chunk_spec():
    return pl.BlockSpec((2, BLK, 128), lambda i: (0, i, 0))


_mm1 = pl.pallas_call(
    _mm1_body,
    grid=(N_NODES // BLK,),
    in_specs=[_deg_spec(), _row_spec(D_IN), _full_spec(D_IN, D_HID)],
    out_specs=[_chunk_spec(), _row_spec(D_HID)],
    out_shape=[
        jax.ShapeDtypeStruct((2, N_NODES, 128), jnp.float32),
        jax.ShapeDtypeStruct((N_NODES, D_HID), jnp.float32),
    ],
)

_mm2 = pl.pallas_call(
    _mm2_body,
    grid=(N_NODES // BLK,),
    in_specs=[
        _deg_spec(),
        _chunk_spec(),
        _row_spec(D_HID),
        _full_spec(1, D_HID),
        _full_spec(D_HID, D_OUT),
    ],
    out_specs=[_row_spec(D_OUT), _row_spec(D_OUT)],
    out_shape=[
        jax.ShapeDtypeStruct((N_NODES, D_OUT), jnp.float32),
        jax.ShapeDtypeStruct((N_NODES, D_OUT), jnp.float32),
    ],
)

_final = pl.pallas_call(
    _final_body,
    grid=(N_NODES // BLK,),
    in_specs=[
        _deg_spec(),
        _chunk_spec(),
        _row_spec(D_OUT),
        _full_spec(1, D_OUT),
    ],
    out_specs=_row_spec(D_OUT),
    out_shape=jax.ShapeDtypeStruct((N_NODES, D_OUT), jnp.float32),
)


def _pad_per_tile(a, n_tiles, pad_value):
    per = a.shape[0] // n_tiles
    per_p = _round_up(per, EB)
    a2 = a.reshape(n_tiles, per)
    a2 = jnp.pad(a2, ((0, 0), (0, per_p - per)), constant_values=pad_value)
    return a2.reshape(-1), per_p


def kernel(x, edge_index, W1, b1, W2, b2):
    src = edge_index[0].astype(jnp.int32)
    dst = edge_index[1].astype(jnp.int32)

    src1, ept1 = _pad_per_tile(src, N_SUB, 0)
    dst1, _ = _pad_per_tile(dst, N_SUB, DUMMY)
    src_l1 = jnp.concatenate([src1, src1 + N_NODES])
    dst_l1 = jnp.concatenate([dst1, dst1])

    src_l2, ept2 = _pad_per_tile(src, N_CORES * N_SUB, 0)
    dst_l2, _ = _pad_per_tile(dst, N_CORES * N_SUB, DUMMY)

    seg1 = _make_segsum(N_CORES * N_NODES, ept1)
    seg2 = _make_segsum(N_NODES, ept2)
    degk = _make_degree(ept2)

    degp = degk(dst_l2).reshape(2, N_NODES, LANES)

    hs_chunks, hself = _mm1(degp, x, W1)
    table1 = hs_chunks.reshape(N_CORES * N_NODES, 128)

    acc1 = seg1(table1, src_l1, dst_l1).reshape(2, N_NODES, 128)

    hs2, hself2 = _mm2(degp, acc1, hself, b1.reshape(1, D_HID), W2)

    acc2 = seg2(hs2, src_l2, dst_l2).reshape(2, N_NODES, 128)

    return _final(degp, acc2, hself2, b2.reshape(1, D_OUT))
